# baseline (speedup 1.0000x reference)
import jax
import jax.numpy as jnp
from jax import lax
from jax.experimental import pallas as pl
from jax.experimental.pallas import tpu as pltpu


def _prep_kernel(s_ref, ws_ref, bs_ref, wsup_ref, bsup_ref,
                 wq_ref, wque_ref, bque_ref,
                 fs16_out, wq16_out, wque16_out, bque_out):
    ns = float(s_ref.shape[1])
    num_pairs, _, hid = wsup_ref.shape

    s_sum = jnp.sum(s_ref[...], axis=1, keepdims=True)
    sup = lax.dot_general(s_sum, ws_ref[...], (((0,), (0,)), ((), ())),
                          preferred_element_type=jnp.float32)
    sup = sup + ns * bs_ref[...]
    for pp in range(num_pairs):
        sl = slice(pp * hid, (pp + 1) * hid)
        f_sp = jnp.dot(sup, wsup_ref[pp],
                       preferred_element_type=jnp.float32) \
            + ns * bsup_ref[pp:pp + 1, :]
        fs16_out[:, sl] = f_sp.astype(jnp.bfloat16)
        wque16_out[:, sl] = wque_ref[pp].astype(jnp.bfloat16)
        bque_out[:, sl] = bque_ref[pp:pp + 1, :]
    wq16_out[...] = wq_ref[...].astype(jnp.bfloat16)


def _stream_kernel(q_ref, wq16_ref, bq_ref, wque16_ref, bque_ref, fs16_ref,
                   out_ref):
    q16 = q_ref[...].astype(jnp.bfloat16)
    que = lax.dot_general(q16, wq16_ref[...], (((0,), (0,)), ((), ())),
                          preferred_element_type=jnp.float32)
    que = que + bq_ref[...]
    f_q = jnp.dot(que.astype(jnp.bfloat16), wque16_ref[...],
                  preferred_element_type=jnp.float32) + bque_ref[...]
    logits = lax.dot_general(fs16_ref[...], f_q.astype(jnp.bfloat16),
                             (((1,), (1,)), ((), ())),
                             preferred_element_type=jnp.float32)
    out_ref[...] = 1.0 / (1.0 + jnp.exp(-logits))


def _pick_tile(nq, max_tile=1024):
    if nq <= max_tile or nq % 128 != 0:
        return nq
    t = max_tile - (max_tile % 128)
    while t >= 128:
        if nq % t == 0:
            return t
        t -= 128
    return nq


def kernel(query_emb, support_emb, wq, bq, ws, bs, wque, bque, wsup, bsup):
    din, nq = query_emb.shape
    _, ns = support_emb.shape
    p, dout, hid = wque.shape
    ph = p * hid

    bq2 = bq.reshape(1, dout)
    bs2 = bs.reshape(1, dout)

    fs16, wq16, wque16, bque_all = pl.pallas_call(
        _prep_kernel,
        out_shape=[
            jax.ShapeDtypeStruct((1, ph), jnp.bfloat16),
            jax.ShapeDtypeStruct((din, dout), jnp.bfloat16),
            jax.ShapeDtypeStruct((dout, ph), jnp.bfloat16),
            jax.ShapeDtypeStruct((1, ph), jnp.float32),
        ],
    )(support_emb, ws, bs2, wsup, bsup, wq, wque, bque)

    tq = _pick_tile(nq)
    n_tiles = nq // tq
    n_cores = 2 if n_tiles % 2 == 0 else 1
    spc = n_tiles // n_cores

    out = pl.pallas_call(
        _stream_kernel,
        out_shape=jax.ShapeDtypeStruct((1, nq), jnp.float32),
        grid=(n_cores, spc),
        in_specs=[
            pl.BlockSpec((din, tq), lambda i, j: (0, i * spc + j)),
            pl.BlockSpec((din, dout), lambda i, j: (0, 0)),
            pl.BlockSpec((1, dout), lambda i, j: (0, 0)),
            pl.BlockSpec((dout, ph), lambda i, j: (0, 0)),
            pl.BlockSpec((1, ph), lambda i, j: (0, 0)),
            pl.BlockSpec((1, ph), lambda i, j: (0, 0)),
        ],
        out_specs=pl.BlockSpec((1, tq), lambda i, j: (0, i * spc + j)),
        compiler_params=pltpu.CompilerParams(
            dimension_semantics=("parallel", "arbitrary")),
    )(query_emb, wq16, bq2, wque16, bque_all, fs16)

    return out.reshape(nq)


# speedup vs baseline: 1.0790x; 1.0790x over previous
import jax
import jax.numpy as jnp
from jax import lax
from jax.experimental import pallas as pl
from jax.experimental.pallas import tpu as pltpu


def _bilinear_kernel(q_ref, s_ref, wq_ref, bq_ref, ws_ref, bs_ref,
                     wque_ref, bque_ref, wsup_ref, bsup_ref,
                     out_ref, wq16_s, wque16_s, bque_s, fs16_s):
    j = pl.program_id(1)
    num_pairs, _, hid = wque_ref.shape

    @pl.when(j == 0)
    def _():
        ns = float(s_ref.shape[1])
        s_sum = jnp.sum(s_ref[...], axis=1, keepdims=True)
        sup = lax.dot_general(s_sum, ws_ref[...], (((0,), (0,)), ((), ())),
                              preferred_element_type=jnp.float32)
        sup = sup + ns * bs_ref[...]
        for pp in range(num_pairs):
            sl = slice(pp * hid, (pp + 1) * hid)
            f_sp = jnp.dot(sup, wsup_ref[pp],
                           preferred_element_type=jnp.float32) \
                + ns * bsup_ref[pp:pp + 1, :]
            fs16_s[:, sl] = f_sp.astype(jnp.bfloat16)
            wque16_s[:, sl] = wque_ref[pp].astype(jnp.bfloat16)
            bque_s[:, sl] = bque_ref[pp:pp + 1, :]
        wq16_s[...] = wq_ref[...].astype(jnp.bfloat16)

    q16 = q_ref[...].astype(jnp.bfloat16)
    que = lax.dot_general(q16, wq16_s[...], (((0,), (0,)), ((), ())),
                          preferred_element_type=jnp.float32)
    que = que + bq_ref[...]
    que16 = que.astype(jnp.bfloat16)

    f_q = jnp.dot(que16, wque16_s[...],
                  preferred_element_type=jnp.float32) + bque_s[...]
    logits = lax.dot_general(fs16_s[...], f_q.astype(jnp.bfloat16),
                             (((1,), (1,)), ((), ())),
                             preferred_element_type=jnp.float32)

    out_ref[...] = 1.0 / (1.0 + jnp.exp(-logits))


def _pick_tile(nq, max_tile=2048):
    if nq <= max_tile or nq % 128 != 0:
        return nq
    t = max_tile - (max_tile % 128)
    while t >= 128:
        if nq % t == 0:
            return t
        t -= 128
    return nq


def kernel(query_emb, support_emb, wq, bq, ws, bs, wque, bque, wsup, bsup):
    din, nq = query_emb.shape
    _, ns = support_emb.shape
    p, dout, hid = wque.shape

    bq2 = bq.reshape(1, dout)
    bs2 = bs.reshape(1, dout)

    tq = _pick_tile(nq)
    n_tiles = nq // tq
    n_cores = 2 if n_tiles % 2 == 0 else 1
    spc = n_tiles // n_cores

    out = pl.pallas_call(
        _bilinear_kernel,
        out_shape=jax.ShapeDtypeStruct((1, nq), jnp.float32),
        grid=(n_cores, spc),
        in_specs=[
            pl.BlockSpec((din, tq), lambda i, j: (0, i * spc + j)),
            pl.BlockSpec((din, ns), lambda i, j: (0, 0)),
            pl.BlockSpec((din, dout), lambda i, j: (0, 0)),
            pl.BlockSpec((1, dout), lambda i, j: (0, 0)),
            pl.BlockSpec((din, dout), lambda i, j: (0, 0)),
            pl.BlockSpec((1, dout), lambda i, j: (0, 0)),
            pl.BlockSpec((p, dout, hid), lambda i, j: (0, 0, 0)),
            pl.BlockSpec((p, hid), lambda i, j: (0, 0)),
            pl.BlockSpec((p, dout, hid), lambda i, j: (0, 0, 0)),
            pl.BlockSpec((p, hid), lambda i, j: (0, 0)),
        ],
        out_specs=pl.BlockSpec((1, tq), lambda i, j: (0, i * spc + j)),
        scratch_shapes=[
            pltpu.VMEM((din, dout), jnp.bfloat16),
            pltpu.VMEM((dout, p * hid), jnp.bfloat16),
            pltpu.VMEM((1, p * hid), jnp.float32),
            pltpu.VMEM((1, p * hid), jnp.bfloat16),
        ],
        compiler_params=pltpu.CompilerParams(
            dimension_semantics=("parallel", "arbitrary")),
    )(query_emb, support_emb, wq, bq2, ws, bs2, wque, bque, wsup, bsup)

    return out.reshape(nq)


# speedup vs baseline: 1.0925x; 1.0125x over previous
import jax
import jax.numpy as jnp
from jax import lax
from jax.experimental import pallas as pl
from jax.experimental.pallas import tpu as pltpu


def _bilinear_kernel(q_ref, s_ref, wq_ref, bq_ref, ws_ref, bs_ref,
                     wque_ref, bque_ref, wsup_ref, bsup_ref,
                     out_ref, wq16_s, wque16_s, bque_s, fs16_s):
    j = pl.program_id(1)
    num_pairs, _, hid = wque_ref.shape

    @pl.when(j == 0)
    def _():
        ns = float(s_ref.shape[1])
        s_sum = jnp.sum(s_ref[...], axis=1, keepdims=True)
        sup = lax.dot_general(s_sum, ws_ref[...], (((0,), (0,)), ((), ())),
                              preferred_element_type=jnp.float32)
        sup = sup + ns * bs_ref[...]
        for pp in range(num_pairs):
            sl = slice(pp * hid, (pp + 1) * hid)
            f_sp = jnp.dot(sup, wsup_ref[pp],
                           preferred_element_type=jnp.float32) \
                + ns * bsup_ref[pp:pp + 1, :]
            fs16_s[:, sl] = f_sp.astype(jnp.bfloat16)
            wque16_s[:, sl] = wque_ref[pp].astype(jnp.bfloat16)
            bque_s[:, sl] = bque_ref[pp:pp + 1, :]
        wq16_s[...] = wq_ref[...].astype(jnp.bfloat16)

    tq = q_ref.shape[1]
    n_sub = max(1, tq // 512)
    w = tq // n_sub
    for h in range(n_sub):
        cols = slice(h * w, (h + 1) * w)
        q16 = q_ref[:, cols].astype(jnp.bfloat16)
        que = lax.dot_general(q16, wq16_s[...], (((0,), (0,)), ((), ())),
                              preferred_element_type=jnp.float32)
        que = que + bq_ref[...]
        f_q = jnp.dot(que.astype(jnp.bfloat16), wque16_s[...],
                      preferred_element_type=jnp.float32) + bque_s[...]
        logits = lax.dot_general(fs16_s[...], f_q.astype(jnp.bfloat16),
                                 (((1,), (1,)), ((), ())),
                                 preferred_element_type=jnp.float32)
        out_ref[:, cols] = 1.0 / (1.0 + jnp.exp(-logits))


def _pick_tile(nq, max_tile=2048):
    if nq <= max_tile or nq % 128 != 0:
        return nq
    t = max_tile - (max_tile % 128)
    while t >= 128:
        if nq % t == 0:
            return t
        t -= 128
    return nq


def kernel(query_emb, support_emb, wq, bq, ws, bs, wque, bque, wsup, bsup):
    din, nq = query_emb.shape
    _, ns = support_emb.shape
    p, dout, hid = wque.shape

    bq2 = bq.reshape(1, dout)
    bs2 = bs.reshape(1, dout)

    tq = _pick_tile(nq)
    n_tiles = nq // tq
    n_cores = 1
    spc = n_tiles // n_cores

    out = pl.pallas_call(
        _bilinear_kernel,
        out_shape=jax.ShapeDtypeStruct((1, nq), jnp.float32),
        grid=(n_cores, spc),
        in_specs=[
            pl.BlockSpec((din, tq), lambda i, j: (0, i * spc + j)),
            pl.BlockSpec((din, ns), lambda i, j: (0, 0)),
            pl.BlockSpec((din, dout), lambda i, j: (0, 0)),
            pl.BlockSpec((1, dout), lambda i, j: (0, 0)),
            pl.BlockSpec((din, dout), lambda i, j: (0, 0)),
            pl.BlockSpec((1, dout), lambda i, j: (0, 0)),
            pl.BlockSpec((p, dout, hid), lambda i, j: (0, 0, 0)),
            pl.BlockSpec((p, hid), lambda i, j: (0, 0)),
            pl.BlockSpec((p, dout, hid), lambda i, j: (0, 0, 0)),
            pl.BlockSpec((p, hid), lambda i, j: (0, 0)),
        ],
        out_specs=pl.BlockSpec((1, tq), lambda i, j: (0, i * spc + j)),
        scratch_shapes=[
            pltpu.VMEM((din, dout), jnp.bfloat16),
            pltpu.VMEM((dout, p * hid), jnp.bfloat16),
            pltpu.VMEM((1, p * hid), jnp.float32),
            pltpu.VMEM((1, p * hid), jnp.bfloat16),
        ],
        compiler_params=pltpu.CompilerParams(
            dimension_semantics=("parallel", "arbitrary")),
    )(query_emb, support_emb, wq, bq2, ws, bs2, wque, bque, wsup, bsup)

    return out.reshape(nq)
